# baseline (speedup 1.0000x reference)
import jax
import jax.numpy as jnp
from jax.experimental import pallas as pl
from jax.experimental.pallas import tpu as pltpu

D = 64
M = 50
BB = 128
RPB = M * BB


def _agree_kernel(ids_ref, aux_ref, len_ref, par_ref, aux2_ref,
                  user_ref, item_ref, group_ref,
                  w1eo_ref, w1ieo_ref, b1_ref, w2b_ref, spread_ref,
                  w1p_ref, b1p_ref, w2p_ref, b2p_ref,
                  o_ref,
                  gi2, heo, hwe, hwo, item_s, grp_s, w_scr, w_up,
                  idx_smem, aux_smem, sem1, sem2):
    cp1 = pltpu.make_async_copy(ids_ref.at[0, 0], idx_smem, sem1)
    cp2 = pltpu.make_async_copy(aux_ref.at[0, 0], aux_smem, sem2)
    cp1.start()
    cp2.start()
    cp1.wait()
    cp2.wait()

    @pl.when(pl.program_id(0) == 0)
    def _():
        gi2[...] = jnp.zeros((RPB, 2 * D), jnp.float32)

    for b in range(BB):
        item_s[pl.ds(b, 1), :] = item_ref[aux_smem[b]]
        grp_s[pl.ds(b, 1), :] = group_ref[aux_smem[BB + b]]

    def mbody(m, carry):
        base = pl.multiple_of(m * BB, BB)
        for b in range(BB):
            gi2[pl.ds(base + b, 1), :] = user_ref[idx_smem[m * BB + b]]
        return carry

    jax.lax.fori_loop(0, aux_smem[2 * BB], mbody, 0)

    heo[...] = jnp.dot(gi2[...], w1eo_ref[...], preferred_element_type=jnp.float32)
    t_eo = jnp.dot(item_s[...], w1ieo_ref[...], preferred_element_type=jnp.float32)
    aux2 = aux2_ref[0]
    t_e = t_eo[:, :16]
    tb = t_e + aux2[:, :16] * (t_eo[:, 16:32] - t_e) + b1_ref[...]

    for m in range(M):
        slab = heo[pl.ds(m * BB, BB), :]
        hwe[:, m * 16:(m + 1) * 16] = jnp.maximum(slab[:, :16] + tb, 0.0)
        hwo[:, m * 16:(m + 1) * 16] = jnp.maximum(slab[:, 16:32] + tb, 0.0)

    s_e = jnp.dot(hwe[...], w2b_ref[...], preferred_element_type=jnp.float32)
    s_o = jnp.dot(hwo[...], w2b_ref[...], preferred_element_type=jnp.float32)
    par = par_ref[0]
    scores = s_e + par * (s_o - s_e)

    lanes = jax.lax.broadcasted_iota(jnp.int32, (BB, BB), 1)
    mask = lanes <= len_ref[0]
    sm = jnp.where(mask, scores, -1e30)
    mx = jnp.max(sm, axis=1, keepdims=True)
    p = jnp.exp(sm - mx)
    wts = p / jnp.sum(p, axis=1, keepdims=True)

    w_o = wts * par
    w_scr[:, :BB] = wts - w_o
    w_scr[:, BB:] = w_o
    w_up[...] = jnp.dot(w_scr[...], spread_ref[...],
                        preferred_element_type=jnp.float32)

    g_pair = jnp.zeros((BB, 2 * D), jnp.float32)
    for m in range(M):
        g_pair = g_pair + (w_up[:, m * BB:(m + 1) * BB]
                           * gi2[pl.ds(m * BB, BB), :])
    g_acc = g_pair[:, :D] + g_pair[:, D:]

    gp = grp_s[...]
    grp_e = gp[:, :D] + aux2[:, D:] * (gp[:, D:] - gp[:, :D])
    ip = item_s[...]
    item_e = ip[:, :D] + aux2[:, :D] * (ip[:, D:] - ip[:, :D])

    g_tot = g_acc + grp_e
    elem = g_tot * item_e

    x = jnp.concatenate([elem, g_tot, item_e], axis=1)
    ph = jnp.maximum(
        jnp.dot(x, w1p_ref[...], preferred_element_type=jnp.float32) + b1p_ref[...],
        0.0)
    y = jnp.dot(ph, w2p_ref[...], preferred_element_type=jnp.float32) + b2p_ref[...]
    o_ref[0] = jax.nn.sigmoid(y[:, :1])


def kernel(group_inputs, item_inputs, member_ids, member_lengths,
           user_table, item_table, group_table,
           att_w1, att_b1, att_w2, att_b2,
           pred_w1, pred_b1, pred_w2, pred_b2):
    B = group_inputs.shape[0]
    NB = B // BB

    user_p = user_table.reshape(-1, 1, 2 * D)
    item_p = item_table.reshape(-1, 1, 2 * D)
    group_p = group_table.reshape(-1, 1, 2 * D)

    perm = jnp.argsort(member_lengths)
    mids = member_ids[perm].astype(jnp.int32)
    iids = item_inputs[perm].astype(jnp.int32)
    gids = group_inputs[perm].astype(jnp.int32)
    lens = member_lengths[perm].astype(jnp.int32)
    n_slabs = lens.reshape(NB, BB).max(axis=1) + 1
    ids_t = ((mids >> 1)
             .reshape(NB, BB, M).transpose(0, 2, 1).reshape(NB, 1, RPB))
    par = jnp.pad((mids & 1).astype(jnp.float32), ((0, 0), (0, BB - M)))
    par = par.reshape(NB, BB, BB)
    aux2 = jnp.concatenate(
        [jnp.broadcast_to((iids & 1).astype(jnp.float32)[:, None], (B, D)),
         jnp.broadcast_to((gids & 1).astype(jnp.float32)[:, None], (B, D))],
        axis=1).reshape(NB, BB, BB)
    aux = jnp.concatenate(
        [(iids >> 1).reshape(NB, 1, BB), (gids >> 1).reshape(NB, 1, BB),
         jnp.broadcast_to(n_slabs[:, None, None], (NB, 1, 8))], axis=2)
    len_r = lens.reshape(NB, BB, 1)

    zeros64 = jnp.zeros((D, 16), jnp.float32)
    w1m_e = jnp.concatenate([att_w1[:D], zeros64], axis=0)
    w1m_o = jnp.concatenate([zeros64, att_w1[:D]], axis=0)
    w1eo = jnp.concatenate([w1m_e, w1m_o], axis=1)
    w1i_e = jnp.concatenate([att_w1[D:], zeros64], axis=0)
    w1i_o = jnp.concatenate([zeros64, att_w1[D:]], axis=0)
    w1ieo = jnp.concatenate([w1i_e, w1i_o], axis=1)
    b1 = att_b1.reshape(1, 16)
    w2b = (jnp.eye(M, dtype=jnp.float32)[:, None, :]
           * att_w2[:, 0][None, :, None]).reshape(M * 16, M)
    w2b = jnp.pad(w2b, ((0, 0), (0, BB - M)))

    srow = jnp.arange(2 * BB, dtype=jnp.int32)[:, None]
    scol = jnp.arange(RPB, dtype=jnp.int32)[None, :]
    mslab = scol // BB
    slane = scol % BB
    spread = (((srow < BB) & (mslab == srow) & (slane < D))
              | ((srow >= BB) & (mslab == srow - BB) & (slane >= D))
              ).astype(jnp.bfloat16)

    w1p = jnp.pad(pred_w1, ((0, 0), (0, BB - 8)))
    b1p = jnp.pad(pred_b1, (0, BB - 8)).reshape(1, BB)
    w2p = jnp.pad(pred_w2, ((0, BB - 8), (0, BB - 1)))
    b2p = jnp.full((1, BB), pred_b2[0], jnp.float32)

    vmem = lambda: pl.BlockSpec(memory_space=pltpu.VMEM)
    out = pl.pallas_call(
        _agree_kernel,
        out_shape=jax.ShapeDtypeStruct((NB, BB, 1), jnp.float32),
        grid=(NB,),
        in_specs=[
            pl.BlockSpec((1, 1, RPB), lambda i: (i, 0, 0)),
            pl.BlockSpec((1, 1, 2 * BB + 8), lambda i: (i, 0, 0)),
            pl.BlockSpec((1, BB, 1), lambda i: (i, 0, 0)),
            pl.BlockSpec((1, BB, BB), lambda i: (i, 0, 0)),
            pl.BlockSpec((1, BB, BB), lambda i: (i, 0, 0)),
            vmem(), vmem(), vmem(),
            vmem(), vmem(), vmem(), vmem(), vmem(),
            vmem(), vmem(), vmem(), vmem(),
        ],
        out_specs=pl.BlockSpec((1, BB, 1), lambda i: (i, 0, 0)),
        scratch_shapes=[
            pltpu.VMEM((RPB, 2 * D), jnp.float32),
            pltpu.VMEM((RPB, 32), jnp.float32),
            pltpu.VMEM((BB, M * 16), jnp.float32),
            pltpu.VMEM((BB, M * 16), jnp.float32),
            pltpu.VMEM((BB, 2 * D), jnp.float32),
            pltpu.VMEM((BB, 2 * D), jnp.float32),
            pltpu.VMEM((BB, 2 * BB), jnp.float32),
            pltpu.VMEM((BB, RPB), jnp.float32),
            pltpu.SMEM((RPB,), jnp.int32),
            pltpu.SMEM((2 * BB + 8,), jnp.int32),
            pltpu.SemaphoreType.DMA,
            pltpu.SemaphoreType.DMA,
        ],
        compiler_params=pltpu.CompilerParams(
            dimension_semantics=("parallel",),
            vmem_limit_bytes=60000 * 1024,
        ),
        name="agree_fused",
    )(ids_t, aux, len_r, par, aux2, user_p, item_p, group_p,
      w1eo, w1ieo, b1, w2b, spread, w1p, b1p, w2p, b2p)
    return jnp.zeros((B, 1), jnp.float32).at[perm].set(out.reshape(B, 1))


# speedup vs baseline: 1.0473x; 1.0473x over previous
import jax
import jax.numpy as jnp
from jax.experimental import pallas as pl
from jax.experimental.pallas import tpu as pltpu

D = 64
M = 50
BB = 128
RPB = M * BB


def _agree_kernel(ids_ref, aux_ref, len_ref, par_ref, aux2_ref,
                  user_ref, item_ref, group_ref,
                  w1eo_ref, w1ieo_ref, b1_ref, w2b_ref, spread_ref,
                  w1p_ref, b1p_ref, w2p_ref, b2p_ref,
                  o_ref,
                  gi2, heo, hwe, hwo, item_s, grp_s, w_scr, w_up,
                  idx_smem, aux_smem, sem1, sem2):
    cp1 = pltpu.make_async_copy(ids_ref.at[0, 0], idx_smem, sem1)
    cp2 = pltpu.make_async_copy(aux_ref.at[0, 0], aux_smem, sem2)
    cp1.start()
    cp2.start()
    cp1.wait()
    cp2.wait()

    @pl.when(pl.program_id(0) == 0)
    def _():
        gi2[...] = jnp.zeros((RPB, 2 * D), jnp.float32)

    for b in range(BB):
        item_s[pl.ds(b, 1), :] = item_ref[aux_smem[b]]
        grp_s[pl.ds(b, 1), :] = group_ref[aux_smem[BB + b]]

    def mbody(m, carry):
        base = pl.multiple_of(m * BB, BB)
        for b in range(BB):
            gi2[pl.ds(base + b, 1), :] = user_ref[idx_smem[m * BB + b]]
        return carry

    jax.lax.fori_loop(0, aux_smem[2 * BB], mbody, 0)

    heo[...] = jnp.dot(gi2[...], w1eo_ref[...], preferred_element_type=jnp.float32)
    t_eo = jnp.dot(item_s[...], w1ieo_ref[...], preferred_element_type=jnp.float32)
    aux2 = aux2_ref[0]
    t_e = t_eo[:, :16]
    tb = t_e + aux2[:, :16] * (t_eo[:, 16:32] - t_e) + b1_ref[...]

    for m in range(M):
        slab = heo[pl.ds(m * BB, BB), :]
        hwe[:, m * 16:(m + 1) * 16] = jnp.maximum(slab[:, :16] + tb, 0.0)
        hwo[:, m * 16:(m + 1) * 16] = jnp.maximum(slab[:, 16:32] + tb, 0.0)

    s_e = jnp.dot(hwe[...], w2b_ref[...], preferred_element_type=jnp.float32)
    s_o = jnp.dot(hwo[...], w2b_ref[...], preferred_element_type=jnp.float32)
    par = par_ref[0]
    scores = s_e + par * (s_o - s_e)

    lanes = jax.lax.broadcasted_iota(jnp.int32, (BB, BB), 1)
    mask = lanes <= len_ref[0]
    sm = jnp.where(mask, scores, -1e30)
    mx = jnp.max(sm, axis=1, keepdims=True)
    p = jnp.exp(sm - mx)
    wts = p / jnp.sum(p, axis=1, keepdims=True)

    w_o = wts * par
    w_scr[:, :BB] = wts - w_o
    w_scr[:, BB:] = w_o
    w_up[...] = jnp.dot(w_scr[...], spread_ref[...],
                        preferred_element_type=jnp.float32)

    g_pair = jnp.zeros((BB, 2 * D), jnp.float32)
    for m in range(M):
        g_pair = g_pair + (w_up[:, m * BB:(m + 1) * BB]
                           * gi2[pl.ds(m * BB, BB), :])
    g_acc = g_pair[:, :D] + g_pair[:, D:]

    gp = grp_s[...]
    grp_e = gp[:, :D] + aux2[:, D:] * (gp[:, D:] - gp[:, :D])
    ip = item_s[...]
    item_e = ip[:, :D] + aux2[:, :D] * (ip[:, D:] - ip[:, :D])

    g_tot = g_acc + grp_e
    elem = g_tot * item_e

    x = jnp.concatenate([elem, g_tot, item_e], axis=1)
    ph = jnp.maximum(
        jnp.dot(x, w1p_ref[...], preferred_element_type=jnp.float32) + b1p_ref[...],
        0.0)
    y = jnp.dot(ph, w2p_ref[...], preferred_element_type=jnp.float32) + b2p_ref[...]
    o_ref[0] = jax.nn.sigmoid(y[:, :1])


def kernel(group_inputs, item_inputs, member_ids, member_lengths,
           user_table, item_table, group_table,
           att_w1, att_b1, att_w2, att_b2,
           pred_w1, pred_b1, pred_w2, pred_b2):
    B = group_inputs.shape[0]
    NB = B // BB

    user_p = user_table.reshape(-1, 1, 2 * D)
    item_p = item_table.reshape(-1, 1, 2 * D)
    group_p = group_table.reshape(-1, 1, 2 * D)

    perm = jnp.argsort(member_lengths)
    mids = member_ids[perm].astype(jnp.int32)
    iids = item_inputs[perm].astype(jnp.int32)
    gids = group_inputs[perm].astype(jnp.int32)
    lens = member_lengths[perm].astype(jnp.int32)
    n_slabs = lens.reshape(NB, BB).max(axis=1) + 1
    ids_t = ((mids >> 1)
             .reshape(NB, BB, M).transpose(0, 2, 1).reshape(NB, 1, RPB))
    par = jnp.pad((mids & 1).astype(jnp.float32), ((0, 0), (0, BB - M)))
    par = par.reshape(NB, BB, BB)
    aux2 = jnp.concatenate(
        [jnp.broadcast_to((iids & 1).astype(jnp.float32)[:, None], (B, D)),
         jnp.broadcast_to((gids & 1).astype(jnp.float32)[:, None], (B, D))],
        axis=1).reshape(NB, BB, BB)
    aux = jnp.concatenate(
        [(iids >> 1).reshape(NB, 1, BB), (gids >> 1).reshape(NB, 1, BB),
         jnp.broadcast_to(n_slabs[:, None, None], (NB, 1, 8))], axis=2)
    len_r = lens.reshape(NB, BB, 1)

    zeros64 = jnp.zeros((D, 16), jnp.float32)
    w1m_e = jnp.concatenate([att_w1[:D], zeros64], axis=0)
    w1m_o = jnp.concatenate([zeros64, att_w1[:D]], axis=0)
    w1eo = jnp.concatenate([w1m_e, w1m_o], axis=1)
    w1i_e = jnp.concatenate([att_w1[D:], zeros64], axis=0)
    w1i_o = jnp.concatenate([zeros64, att_w1[D:]], axis=0)
    w1ieo = jnp.concatenate([w1i_e, w1i_o], axis=1)
    b1 = att_b1.reshape(1, 16)
    w2b = (jnp.eye(M, dtype=jnp.float32)[:, None, :]
           * att_w2[:, 0][None, :, None]).reshape(M * 16, M)
    w2b = jnp.pad(w2b, ((0, 0), (0, BB - M)))

    srow = jnp.arange(2 * BB, dtype=jnp.int32)[:, None]
    scol = jnp.arange(RPB, dtype=jnp.int32)[None, :]
    mslab = scol // BB
    slane = scol % BB
    spread = (((srow < BB) & (mslab == srow) & (slane < D))
              | ((srow >= BB) & (mslab == srow - BB) & (slane >= D))
              ).astype(jnp.bfloat16)

    w1p = jnp.pad(pred_w1, ((0, 0), (0, BB - 8)))
    b1p = jnp.pad(pred_b1, (0, BB - 8)).reshape(1, BB)
    w2p = jnp.pad(pred_w2, ((0, BB - 8), (0, BB - 1)))
    b2p = jnp.full((1, BB), pred_b2[0], jnp.float32)

    vmem = lambda: pl.BlockSpec(memory_space=pltpu.VMEM)
    out = pl.pallas_call(
        _agree_kernel,
        out_shape=jax.ShapeDtypeStruct((NB, BB, 1), jnp.float32),
        grid=(NB,),
        in_specs=[
            pl.BlockSpec((1, 1, RPB), lambda i: (i, 0, 0)),
            pl.BlockSpec((1, 1, 2 * BB + 8), lambda i: (i, 0, 0)),
            pl.BlockSpec((1, BB, 1), lambda i: (i, 0, 0)),
            pl.BlockSpec((1, BB, BB), lambda i: (i, 0, 0)),
            pl.BlockSpec((1, BB, BB), lambda i: (i, 0, 0)),
            vmem(), vmem(), vmem(),
            vmem(), vmem(), vmem(), vmem(), vmem(),
            vmem(), vmem(), vmem(), vmem(),
        ],
        out_specs=pl.BlockSpec((1, BB, 1), lambda i: (i, 0, 0)),
        scratch_shapes=[
            pltpu.VMEM((RPB, 2 * D), jnp.float32),
            pltpu.VMEM((RPB, 32), jnp.float32),
            pltpu.VMEM((BB, M * 16), jnp.float32),
            pltpu.VMEM((BB, M * 16), jnp.float32),
            pltpu.VMEM((BB, 2 * D), jnp.float32),
            pltpu.VMEM((BB, 2 * D), jnp.float32),
            pltpu.VMEM((BB, 2 * BB), jnp.float32),
            pltpu.VMEM((BB, RPB), jnp.float32),
            pltpu.SMEM((RPB,), jnp.int32),
            pltpu.SMEM((2 * BB + 8,), jnp.int32),
            pltpu.SemaphoreType.DMA,
            pltpu.SemaphoreType.DMA,
        ],
        compiler_params=pltpu.CompilerParams(
            dimension_semantics=("parallel",),
            vmem_limit_bytes=60000 * 1024,
        ),
        name="agree_fused",
    )(ids_t, aux, len_r, par, aux2, user_p, item_p, group_p,
      w1eo, w1ieo, b1, w2b, spread, w1p, b1p, w2p, b2p)
    inv_perm = jnp.argsort(perm)
    return out.reshape(B, 1)[inv_perm]


# speedup vs baseline: 1.0610x; 1.0131x over previous
import jax
import jax.numpy as jnp
from jax.experimental import pallas as pl
from jax.experimental.pallas import tpu as pltpu

D = 64
M = 50
BB = 128
RPB = M * BB


def _agree_kernel(ids_ref, aux_ref, len_ref, par_ref, aux2_ref,
                  user_ref, item_ref, group_ref,
                  w1eo_ref, w1ieo_ref, b1_ref, w2b_ref, spread_ref, sp16_ref,
                  w1p_ref, b1p_ref, w2p_ref, b2p_ref,
                  o_ref,
                  gi2, heo, hwe, hwo, item_s, grp_s, w_scr, w_up, tbr,
                  idx_smem, aux_smem, sem1, sem2):
    cp1 = pltpu.make_async_copy(ids_ref.at[0, 0], idx_smem, sem1)
    cp2 = pltpu.make_async_copy(aux_ref.at[0, 0], aux_smem, sem2)
    cp1.start()
    cp2.start()
    cp1.wait()
    cp2.wait()

    @pl.when(pl.program_id(0) == 0)
    def _():
        gi2[...] = jnp.zeros((RPB, 2 * D), jnp.float32)

    for b in range(BB):
        item_s[pl.ds(b, 1), :] = item_ref[aux_smem[b]]
        grp_s[pl.ds(b, 1), :] = group_ref[aux_smem[BB + b]]

    def mbody(m, carry):
        base = pl.multiple_of(m * BB, BB)
        for b in range(BB):
            gi2[pl.ds(base + b, 1), :] = user_ref[idx_smem[m * BB + b]]
        return carry

    jax.lax.fori_loop(0, aux_smem[2 * BB], mbody, 0)

    heo[...] = jnp.dot(gi2[...], w1eo_ref[...], preferred_element_type=jnp.float32)
    t_eo = jnp.dot(item_s[...], w1ieo_ref[...], preferred_element_type=jnp.float32)
    aux2 = aux2_ref[0]
    t_e = t_eo[:, :16]
    tb = t_e + aux2[:, :16] * (t_eo[:, 16:32] - t_e) + b1_ref[...]

    for m in range(M):
        slab = heo[pl.ds(m * BB, BB), :]
        hwe[:, m * 16:(m + 1) * 16] = slab[:, :16]
        hwo[:, m * 16:(m + 1) * 16] = slab[:, 16:32]
    tbr[...] = jnp.dot(tb, sp16_ref[...], preferred_element_type=jnp.float32)
    hwe[...] = jnp.maximum(hwe[...] + tbr[...], 0.0)
    hwo[...] = jnp.maximum(hwo[...] + tbr[...], 0.0)

    s_e = jnp.dot(hwe[...], w2b_ref[...], preferred_element_type=jnp.float32)
    s_o = jnp.dot(hwo[...], w2b_ref[...], preferred_element_type=jnp.float32)
    par = par_ref[0]
    scores = s_e + par * (s_o - s_e)

    lanes = jax.lax.broadcasted_iota(jnp.int32, (BB, BB), 1)
    mask = lanes <= len_ref[0]
    sm = jnp.where(mask, scores, -1e30)
    mx = jnp.max(sm, axis=1, keepdims=True)
    p = jnp.exp(sm - mx)
    wts = p / jnp.sum(p, axis=1, keepdims=True)

    w_o = wts * par
    w_scr[:, :BB] = wts - w_o
    w_scr[:, BB:] = w_o
    w_up[...] = jnp.dot(w_scr[...], spread_ref[...],
                        preferred_element_type=jnp.float32)

    g_pair = jnp.zeros((BB, 2 * D), jnp.float32)
    for m in range(M):
        g_pair = g_pair + (w_up[:, m * BB:(m + 1) * BB]
                           * gi2[pl.ds(m * BB, BB), :])
    g_acc = g_pair[:, :D] + g_pair[:, D:]

    gp = grp_s[...]
    grp_e = gp[:, :D] + aux2[:, D:] * (gp[:, D:] - gp[:, :D])
    ip = item_s[...]
    item_e = ip[:, :D] + aux2[:, :D] * (ip[:, D:] - ip[:, :D])

    g_tot = g_acc + grp_e
    elem = g_tot * item_e

    x = jnp.concatenate([elem, g_tot, item_e], axis=1)
    ph = jnp.maximum(
        jnp.dot(x, w1p_ref[...], preferred_element_type=jnp.float32) + b1p_ref[...],
        0.0)
    y = jnp.dot(ph, w2p_ref[...], preferred_element_type=jnp.float32) + b2p_ref[...]
    o_ref[0] = jax.nn.sigmoid(y[:, :1])


def kernel(group_inputs, item_inputs, member_ids, member_lengths,
           user_table, item_table, group_table,
           att_w1, att_b1, att_w2, att_b2,
           pred_w1, pred_b1, pred_w2, pred_b2):
    B = group_inputs.shape[0]
    NB = B // BB

    user_p = user_table.reshape(-1, 1, 2 * D)
    item_p = item_table.reshape(-1, 1, 2 * D)
    group_p = group_table.reshape(-1, 1, 2 * D)

    perm = jnp.argsort(member_lengths)
    mids = member_ids[perm].astype(jnp.int32)
    iids = item_inputs[perm].astype(jnp.int32)
    gids = group_inputs[perm].astype(jnp.int32)
    lens = member_lengths[perm].astype(jnp.int32)
    n_slabs = lens.reshape(NB, BB).max(axis=1) + 1
    ids_t = ((mids >> 1)
             .reshape(NB, BB, M).transpose(0, 2, 1).reshape(NB, 1, RPB))
    par = jnp.pad((mids & 1).astype(jnp.float32), ((0, 0), (0, BB - M)))
    par = par.reshape(NB, BB, BB)
    aux2 = jnp.concatenate(
        [jnp.broadcast_to((iids & 1).astype(jnp.float32)[:, None], (B, D)),
         jnp.broadcast_to((gids & 1).astype(jnp.float32)[:, None], (B, D))],
        axis=1).reshape(NB, BB, BB)
    aux = jnp.concatenate(
        [(iids >> 1).reshape(NB, 1, BB), (gids >> 1).reshape(NB, 1, BB),
         jnp.broadcast_to(n_slabs[:, None, None], (NB, 1, 8))], axis=2)
    len_r = lens.reshape(NB, BB, 1)

    zeros64 = jnp.zeros((D, 16), jnp.float32)
    w1m_e = jnp.concatenate([att_w1[:D], zeros64], axis=0)
    w1m_o = jnp.concatenate([zeros64, att_w1[:D]], axis=0)
    w1eo = jnp.concatenate([w1m_e, w1m_o], axis=1)
    w1i_e = jnp.concatenate([att_w1[D:], zeros64], axis=0)
    w1i_o = jnp.concatenate([zeros64, att_w1[D:]], axis=0)
    w1ieo = jnp.concatenate([w1i_e, w1i_o], axis=1)
    b1 = att_b1.reshape(1, 16)
    w2b = (jnp.eye(M, dtype=jnp.float32)[:, None, :]
           * att_w2[:, 0][None, :, None]).reshape(M * 16, M)
    w2b = jnp.pad(w2b, ((0, 0), (0, BB - M)))

    srow = jnp.arange(2 * BB, dtype=jnp.int32)[:, None]
    scol = jnp.arange(RPB, dtype=jnp.int32)[None, :]
    mslab = scol // BB
    slane = scol % BB
    spread = (((srow < BB) & (mslab == srow) & (slane < D))
              | ((srow >= BB) & (mslab == srow - BB) & (slane >= D))
              ).astype(jnp.bfloat16)

    sp16 = jnp.tile(jnp.eye(16, dtype=jnp.float32), (1, M))

    w1p = jnp.pad(pred_w1, ((0, 0), (0, BB - 8)))
    b1p = jnp.pad(pred_b1, (0, BB - 8)).reshape(1, BB)
    w2p = jnp.pad(pred_w2, ((0, BB - 8), (0, BB - 1)))
    b2p = jnp.full((1, BB), pred_b2[0], jnp.float32)

    vmem = lambda: pl.BlockSpec(memory_space=pltpu.VMEM)
    out = pl.pallas_call(
        _agree_kernel,
        out_shape=jax.ShapeDtypeStruct((NB, BB, 1), jnp.float32),
        grid=(NB,),
        in_specs=[
            pl.BlockSpec((1, 1, RPB), lambda i: (i, 0, 0)),
            pl.BlockSpec((1, 1, 2 * BB + 8), lambda i: (i, 0, 0)),
            pl.BlockSpec((1, BB, 1), lambda i: (i, 0, 0)),
            pl.BlockSpec((1, BB, BB), lambda i: (i, 0, 0)),
            pl.BlockSpec((1, BB, BB), lambda i: (i, 0, 0)),
            vmem(), vmem(), vmem(),
            vmem(), vmem(), vmem(), vmem(), vmem(), vmem(),
            vmem(), vmem(), vmem(), vmem(),
        ],
        out_specs=pl.BlockSpec((1, BB, 1), lambda i: (i, 0, 0)),
        scratch_shapes=[
            pltpu.VMEM((RPB, 2 * D), jnp.float32),
            pltpu.VMEM((RPB, 32), jnp.float32),
            pltpu.VMEM((BB, M * 16), jnp.float32),
            pltpu.VMEM((BB, M * 16), jnp.float32),
            pltpu.VMEM((BB, 2 * D), jnp.float32),
            pltpu.VMEM((BB, 2 * D), jnp.float32),
            pltpu.VMEM((BB, 2 * BB), jnp.float32),
            pltpu.VMEM((BB, RPB), jnp.float32),
            pltpu.VMEM((BB, M * 16), jnp.float32),
            pltpu.SMEM((RPB,), jnp.int32),
            pltpu.SMEM((2 * BB + 8,), jnp.int32),
            pltpu.SemaphoreType.DMA,
            pltpu.SemaphoreType.DMA,
        ],
        compiler_params=pltpu.CompilerParams(
            dimension_semantics=("parallel",),
            vmem_limit_bytes=60000 * 1024,
        ),
        name="agree_fused",
    )(ids_t, aux, len_r, par, aux2, user_p, item_p, group_p,
      w1eo, w1ieo, b1, w2b, spread, sp16, w1p, b1p, w2p, b2p)
    inv_perm = jnp.argsort(perm)
    return out.reshape(B, 1)[inv_perm]
